# baseline (speedup 1.0000x reference)
import functools

import jax
import jax.numpy as jnp
from jax.experimental import pallas as pl
from jax.experimental.pallas import tpu as pltpu

NUM_CLASSES = 81
THRESHOLD = 0.5
NEGPOS_RATIO = 3
VAR0 = 0.1
VAR1 = 0.2


def _match(tx1, ty1, tx2, ty2, px1, py1, px2, py2, T, P):
    ix = jnp.maximum(jnp.minimum(tx2, px2) - jnp.maximum(tx1, px1), 0.0)
    iy = jnp.maximum(jnp.minimum(ty2, py2) - jnp.maximum(ty1, py1), 0.0)
    inter = ix * iy
    area_t = (tx2 - tx1) * (ty2 - ty1)
    area_p = (px2 - px1) * (py2 - py1)
    ov = inter / (area_t + area_p - inter)

    iota_p = jax.lax.broadcasted_iota(jnp.int32, (T, P), 1)
    iota_t = jax.lax.broadcasted_iota(jnp.int32, (T, P), 0)

    bp_max = jnp.max(ov, axis=1, keepdims=True)
    bp_idx = jnp.min(jnp.where(ov >= bp_max, iota_p, P), axis=1,
                     keepdims=True)

    bt_max = jnp.max(ov, axis=0, keepdims=True)
    bt_idx = jnp.min(jnp.where(ov >= bt_max, iota_t, T), axis=0,
                     keepdims=True)

    claimed = iota_p == bp_idx
    forced = jnp.max(jnp.where(claimed, iota_t, -1), axis=0,
                     keepdims=True)
    bto = jnp.where(forced >= 0, 2.0, bt_max)
    bti = jnp.where(forced >= 0, forced, bt_idx)
    return bto, bti


def _gather_cols(box_cols, onehot):
    return jax.lax.dot_general(
        box_cols, onehot, (((0,), (0,)), ((), ())),
        preferred_element_type=jnp.float32)


def _image_body(loc_ref, conf_ref, priors_ref, tgt_ref, det_ref,
                scal_ref, mine_ref, *, P, T, C):
    prT = priors_ref[...]
    cx = prT[0:1, :]
    cy = prT[1:2, :]
    pw = prT[2:3, :]
    ph = prT[3:4, :]
    px1 = cx - pw * 0.5
    py1 = cy - ph * 0.5
    px2 = cx + pw * 0.5
    py2 = cy + ph * 0.5

    tgt = tgt_ref[0]
    det = det_ref[0]

    bto_t, bti_t = _match(tgt[:, 0:1], tgt[:, 1:2], tgt[:, 2:3], tgt[:, 3:4],
                          px1, py1, px2, py2, T, P)
    pos = bto_t >= THRESHOLD

    iota_t_col = jax.lax.broadcasted_iota(jnp.int32, (T, P), 0)
    oh_t = (iota_t_col == bti_t).astype(jnp.float32)
    m4 = _gather_cols(tgt[:, 0:4], oh_t)
    mx1 = m4[0:1, :]
    my1 = m4[1:2, :]
    mx2 = m4[2:3, :]
    my2 = m4[3:4, :]

    g_cx = ((mx1 + mx2) * 0.5 - cx) / (VAR0 * pw)
    g_cy = ((my1 + my2) * 0.5 - cy) / (VAR0 * ph)
    g_w = jnp.log((mx2 - mx1) / pw) / VAR1
    g_h = jnp.log((my2 - my1) / ph) / VAR1

    locT = jnp.swapaxes(loc_ref[0], 0, 1)
    d0 = locT[0:1, :] - g_cx
    d1 = locT[1:2, :] - g_cy
    d2 = locT[2:3, :] - g_w
    d3 = locT[3:4, :] - g_h

    def sl1(d):
        ad = jnp.abs(d)
        return jnp.where(ad < 1.0, 0.5 * d * d, ad - 0.5)

    posf = pos.astype(jnp.float32)
    loss_l = jnp.sum((sl1(d0) + sl1(d1) + sl1(d2) + sl1(d3)) * posf)

    bto_d, bti_d = _match(det[:, 0:1], det[:, 1:2], det[:, 2:3], det[:, 3:4],
                          px1, py1, px2, py2, T, P)
    oh_d = (iota_t_col == bti_d).astype(jnp.float32)
    lab_g = _gather_cols(det[:, 4:5], oh_d)
    det_pos = bto_d >= THRESHOLD
    det_conf = jnp.where(det_pos, lab_g.astype(jnp.int32) + 1, 0)

    conf = conf_ref[0]
    row_max = jnp.max(conf, axis=1, keepdims=True)
    e = jnp.exp(conf - row_max)
    ones_c = jnp.ones((C, 1), jnp.float32)
    sum_e = jax.lax.dot_general(e, ones_c, (((1,), (0,)), ((), ())),
                                preferred_element_type=jnp.float32)
    iota_c = jax.lax.broadcasted_iota(jnp.int32, (P, C), 1)
    oh_c = (iota_c == det_conf[0][:, None]).astype(jnp.float32)
    gathered = jax.lax.dot_general(conf * oh_c, ones_c,
                                   (((1,), (0,)), ((), ())),
                                   preferred_element_type=jnp.float32)
    ce = (jnp.log(sum_e) + row_max - gathered)[:, 0][None, :]

    num_pos = jnp.sum(det_pos.astype(jnp.int32))
    ce_pos_sum = jnp.sum(jnp.where(det_pos, ce, 0.0))
    row = pl.program_id(0) % 8
    mine_ref[pl.ds(row, 1), :] = jnp.where(det_pos, 0.0, ce)

    scal_ref[0] = jnp.stack([
        jnp.full((128,), loss_l, jnp.float32),
        jnp.full((128,), ce_pos_sum, jnp.float32),
        jnp.full((128,), num_pos.astype(jnp.float32), jnp.float32),
    ])


def _mine_body(scal_ref, mine_ref, out_ref, *, B, P):
    mine = mine_ref[...]
    scal = scal_ref[...]
    npos = scal[:, 2, 0:1]
    k = jnp.minimum(jnp.int32(NEGPOS_RATIO) * npos.astype(jnp.int32),
                    P - 1)
    vbits = jax.lax.bitcast_convert_type(mine, jnp.int32)

    def bs_step(_, carry):
        lo, hi = carry
        mid = lo + jax.lax.div(hi - lo, 2)
        cnt = jnp.sum((vbits > mid).astype(jnp.int32), axis=1,
                      keepdims=True)
        take_lo = cnt >= k
        return (jnp.where(take_lo, mid, lo), jnp.where(take_lo, hi, mid))

    lo0 = jnp.full((B, 1), -1, jnp.int32)
    hi0 = jnp.full((B, 1), 0x7F7FFFFF, jnp.int32)
    _, hi = jax.lax.fori_loop(0, 31, bs_step, (lo0, hi0))
    thr = jax.lax.bitcast_convert_type(hi, jnp.float32)
    gt = vbits > hi
    cnt_gt = jnp.sum(gt.astype(jnp.int32), axis=1, keepdims=True)
    top = (jnp.sum(jnp.where(gt, mine, 0.0), axis=1, keepdims=True)
           + (k - cnt_gt).astype(jnp.float32) * thr)
    top = jnp.where(k > 0, top, 0.0)

    n = jnp.sum(npos)
    loss_l = jnp.sum(scal[:, 0, 0:1]) / n
    loss_c = jnp.sum(scal[:, 1, 0:1] + top) / n
    out_ref[...] = jnp.stack([jnp.full((128,), loss_l, jnp.float32),
                              jnp.full((128,), loss_c, jnp.float32)])


@jax.jit
def kernel(loc_data, conf_data, priors, targets, det):
    B, P, _ = loc_data.shape
    C = conf_data.shape[-1]
    T = targets.shape[1]

    priors_t = priors.T

    body = functools.partial(_image_body, P=P, T=T, C=C)
    scal, mine = pl.pallas_call(
        body,
        grid=(B,),
        in_specs=[
            pl.BlockSpec((1, P, 4), lambda b: (b, 0, 0)),
            pl.BlockSpec((1, P, C), lambda b: (b, 0, 0)),
            pl.BlockSpec((4, P), lambda b: (0, 0)),
            pl.BlockSpec((1, T, 5), lambda b: (b, 0, 0)),
            pl.BlockSpec((1, T, 5), lambda b: (b, 0, 0)),
        ],
        out_specs=[
            pl.BlockSpec((1, 3, 128), lambda b: (b, 0, 0)),
            pl.BlockSpec((8, P), lambda b: (b // 8, 0)),
        ],
        out_shape=[
            jax.ShapeDtypeStruct((B, 3, 128), jnp.float32),
            jax.ShapeDtypeStruct((B, P), jnp.float32),
        ],
        compiler_params=pltpu.CompilerParams(
            dimension_semantics=("arbitrary",),
        ),
    )(loc_data, conf_data, priors_t, targets, det)

    out = pl.pallas_call(
        functools.partial(_mine_body, B=B, P=P),
        out_shape=jax.ShapeDtypeStruct((2, 128), jnp.float32),
    )(scal, mine)

    return (out[0, 0], out[1, 0])


# speedup vs baseline: 1.2984x; 1.2984x over previous
import functools

import jax
import jax.numpy as jnp
from jax.experimental import pallas as pl
from jax.experimental.pallas import tpu as pltpu

NUM_CLASSES = 81
THRESHOLD = 0.5
NEGPOS_RATIO = 3
VAR0 = 0.1
VAR1 = 0.2


def _match(tx1, ty1, tx2, ty2, px1, py1, px2, py2, T, P):
    ix = jnp.maximum(jnp.minimum(tx2, px2) - jnp.maximum(tx1, px1), 0.0)
    iy = jnp.maximum(jnp.minimum(ty2, py2) - jnp.maximum(ty1, py1), 0.0)
    inter = ix * iy
    area_t = (tx2 - tx1) * (ty2 - ty1)
    area_p = (px2 - px1) * (py2 - py1)
    ov = inter / (area_t + area_p - inter)

    iota_p = jax.lax.broadcasted_iota(jnp.int32, (T, P), 1)
    iota_t = jax.lax.broadcasted_iota(jnp.int32, (T, P), 0)

    bp_max = jnp.max(ov, axis=1, keepdims=True)
    bp_idx = jnp.min(jnp.where(ov >= bp_max, iota_p, P), axis=1,
                     keepdims=True)

    bt_max = jnp.max(ov, axis=0, keepdims=True)
    bt_idx = jnp.min(jnp.where(ov >= bt_max, iota_t, T), axis=0,
                     keepdims=True)

    claimed = iota_p == bp_idx
    forced = jnp.max(jnp.where(claimed, iota_t, -1), axis=0,
                     keepdims=True)
    bto = jnp.where(forced >= 0, 2.0, bt_max)
    bti = jnp.where(forced >= 0, forced, bt_idx)
    return bto, bti


def _gather_cols(box_cols, onehot):
    return jax.lax.dot_general(
        box_cols, onehot, (((0,), (0,)), ((), ())),
        preferred_element_type=jnp.float32)


def _image_body(loc_ref, conf_ref, priors_ref, tgt_ref, det_ref,
                scal_ref, mine_ref, *, P, T, C):
    prT = priors_ref[...]
    cx = prT[0:1, :]
    cy = prT[1:2, :]
    pw = prT[2:3, :]
    ph = prT[3:4, :]
    px1 = cx - pw * 0.5
    py1 = cy - ph * 0.5
    px2 = cx + pw * 0.5
    py2 = cy + ph * 0.5

    tgt = tgt_ref[0]
    det = det_ref[0]

    bto_t, bti_t = _match(tgt[:, 0:1], tgt[:, 1:2], tgt[:, 2:3], tgt[:, 3:4],
                          px1, py1, px2, py2, T, P)
    pos = bto_t >= THRESHOLD

    iota_t_col = jax.lax.broadcasted_iota(jnp.int32, (T, P), 0)
    oh_t = (iota_t_col == bti_t).astype(jnp.float32)
    m4 = _gather_cols(tgt[:, 0:4], oh_t)
    mx1 = m4[0:1, :]
    my1 = m4[1:2, :]
    mx2 = m4[2:3, :]
    my2 = m4[3:4, :]

    g_cx = ((mx1 + mx2) * 0.5 - cx) / (VAR0 * pw)
    g_cy = ((my1 + my2) * 0.5 - cy) / (VAR0 * ph)
    g_w = jnp.log((mx2 - mx1) / pw) / VAR1
    g_h = jnp.log((my2 - my1) / ph) / VAR1

    locT = loc_ref[0]
    d0 = locT[0:1, :] - g_cx
    d1 = locT[1:2, :] - g_cy
    d2 = locT[2:3, :] - g_w
    d3 = locT[3:4, :] - g_h

    def sl1(d):
        ad = jnp.abs(d)
        return jnp.where(ad < 1.0, 0.5 * d * d, ad - 0.5)

    posf = pos.astype(jnp.float32)
    loss_l = jnp.sum((sl1(d0) + sl1(d1) + sl1(d2) + sl1(d3)) * posf)

    bto_d, bti_d = _match(det[:, 0:1], det[:, 1:2], det[:, 2:3], det[:, 3:4],
                          px1, py1, px2, py2, T, P)
    oh_d = (iota_t_col == bti_d).astype(jnp.float32)
    lab_g = _gather_cols(det[:, 4:5], oh_d)
    det_pos = bto_d >= THRESHOLD
    det_conf = jnp.where(det_pos, lab_g.astype(jnp.int32) + 1, 0)

    conf = conf_ref[0]
    row_max = jnp.max(conf, axis=1, keepdims=True)
    e = jnp.exp(conf - row_max)
    ones_c = jnp.ones((C, 1), jnp.float32)
    sum_e = jax.lax.dot_general(e, ones_c, (((1,), (0,)), ((), ())),
                                preferred_element_type=jnp.float32)
    iota_c = jax.lax.broadcasted_iota(jnp.int32, (P, C), 1)
    oh_c = (iota_c == det_conf[0][:, None]).astype(jnp.float32)
    gathered = jax.lax.dot_general(conf * oh_c, ones_c,
                                   (((1,), (0,)), ((), ())),
                                   preferred_element_type=jnp.float32)
    ce = (jnp.log(sum_e) + row_max - gathered)[:, 0][None, :]

    num_pos = jnp.sum(det_pos.astype(jnp.int32))
    ce_pos_sum = jnp.sum(jnp.where(det_pos, ce, 0.0))
    row = pl.program_id(0) % 8
    mine_ref[pl.ds(row, 1), :] = jnp.where(det_pos, 0.0, ce)

    scal_ref[0] = jnp.stack([
        jnp.full((128,), loss_l, jnp.float32),
        jnp.full((128,), ce_pos_sum, jnp.float32),
        jnp.full((128,), num_pos.astype(jnp.float32), jnp.float32),
    ])


def _mine_body(scal_ref, mine_ref, out_ref, *, B, P):
    mine = mine_ref[...]
    scal = scal_ref[...]
    npos = scal[:, 2, 0:1]
    k = jnp.minimum(jnp.int32(NEGPOS_RATIO) * npos.astype(jnp.int32),
                    P - 1)
    vbits = jax.lax.bitcast_convert_type(mine, jnp.int32)

    def bs_step(_, carry):
        lo, hi = carry
        mid = lo + jax.lax.div(hi - lo, 2)
        cnt = jnp.sum((vbits > mid).astype(jnp.int32), axis=1,
                      keepdims=True)
        take_lo = cnt >= k
        return (jnp.where(take_lo, mid, lo), jnp.where(take_lo, hi, mid))

    lo0 = jnp.full((B, 1), -1, jnp.int32)
    hi0 = jnp.full((B, 1), 0x7F7FFFFF, jnp.int32)
    _, hi = jax.lax.fori_loop(0, 31, bs_step, (lo0, hi0))
    thr = jax.lax.bitcast_convert_type(hi, jnp.float32)
    gt = vbits > hi
    cnt_gt = jnp.sum(gt.astype(jnp.int32), axis=1, keepdims=True)
    top = (jnp.sum(jnp.where(gt, mine, 0.0), axis=1, keepdims=True)
           + (k - cnt_gt).astype(jnp.float32) * thr)
    top = jnp.where(k > 0, top, 0.0)

    n = jnp.sum(npos)
    loss_l = jnp.sum(scal[:, 0, 0:1]) / n
    loss_c = jnp.sum(scal[:, 1, 0:1] + top) / n
    out_ref[...] = jnp.stack([jnp.full((128,), loss_l, jnp.float32),
                              jnp.full((128,), loss_c, jnp.float32)])


@jax.jit
def kernel(loc_data, conf_data, priors, targets, det):
    B, P, _ = loc_data.shape
    C = conf_data.shape[-1]
    T = targets.shape[1]

    loc_t = jnp.swapaxes(loc_data, 1, 2)
    priors_t = priors.T

    body = functools.partial(_image_body, P=P, T=T, C=C)
    scal, mine = pl.pallas_call(
        body,
        grid=(B,),
        in_specs=[
            pl.BlockSpec((1, 4, P), lambda b: (b, 0, 0)),
            pl.BlockSpec((1, P, C), lambda b: (b, 0, 0)),
            pl.BlockSpec((4, P), lambda b: (0, 0)),
            pl.BlockSpec((1, T, 5), lambda b: (b, 0, 0)),
            pl.BlockSpec((1, T, 5), lambda b: (b, 0, 0)),
        ],
        out_specs=[
            pl.BlockSpec((1, 3, 128), lambda b: (b, 0, 0)),
            pl.BlockSpec((8, P), lambda b: (b // 8, 0)),
        ],
        out_shape=[
            jax.ShapeDtypeStruct((B, 3, 128), jnp.float32),
            jax.ShapeDtypeStruct((B, P), jnp.float32),
        ],
        compiler_params=pltpu.CompilerParams(
            dimension_semantics=("arbitrary",),
        ),
    )(loc_t, conf_data, priors_t, targets, det)

    out = pl.pallas_call(
        functools.partial(_mine_body, B=B, P=P),
        out_shape=jax.ShapeDtypeStruct((2, 128), jnp.float32),
    )(scal, mine)

    return (out[0, 0], out[1, 0])
